# Initial kernel scaffold; baseline (speedup 1.0000x reference)
import jax
import jax.numpy as jnp
from jax.experimental import pallas as pl


def kernel(encoded_row, wafer_types, loadport_mask, W):
    raise NotImplementedError("write your pallas kernel here")


# speedup vs baseline: 1.2768x; 1.2768x over previous
import functools

import jax
import jax.numpy as jnp
from jax import lax
from jax.experimental import pallas as pl

B, R, W_CNT, D = 4096, 100, 200, 128
BB = 128
JC = 8


def _body(wt_ref, m_ref, enc_ref, w_ref, out_ref):
    wt = wt_ref[...]
    m = m_ref[...]

    iota_r = lax.broadcasted_iota(jnp.int32, (BB, JC, R), 2)

    counts = jnp.zeros((BB, R), jnp.float32)
    for c in range(W_CNT // JC):
        wtc = wt[:, c * JC:(c + 1) * JC]
        mc = m[:, c * JC:(c + 1) * JC]
        oh = (wtc[:, :, None] == iota_r).astype(jnp.float32) * mc[:, :, None]
        counts = counts + oh.sum(axis=1)

    pooled = jnp.zeros((BB, D), jnp.float32)
    for c in range(pl.cdiv(R, JC)):
        lo, hi = c * JC, min((c + 1) * JC, R)
        encc = enc_ref[:, lo:hi, :]
        cc = counts[:, lo:hi]
        pooled = pooled + (encc * cc[:, :, None]).sum(axis=1)

    out_ref[...] = jnp.dot(pooled, w_ref[...].T,
                           preferred_element_type=jnp.float32)


@jax.jit
def kernel(encoded_row, wafer_types, loadport_mask, W):
    wt = wafer_types.astype(jnp.int32)
    m = loadport_mask.astype(jnp.float32)
    grid = (B // BB,)
    return pl.pallas_call(
        _body,
        grid=grid,
        in_specs=[
            pl.BlockSpec((BB, W_CNT), lambda i: (i, 0)),
            pl.BlockSpec((BB, W_CNT), lambda i: (i, 0)),
            pl.BlockSpec((BB, R, D), lambda i: (i, 0, 0)),
            pl.BlockSpec((D, D), lambda i: (0, 0)),
        ],
        out_specs=pl.BlockSpec((BB, D), lambda i: (i, 0)),
        out_shape=jax.ShapeDtypeStruct((B, D), jnp.float32),
    )(wt, m, encoded_row, W)


# speedup vs baseline: 1.7310x; 1.3557x over previous
import functools

import jax
import jax.numpy as jnp
from jax import lax
from jax.experimental import pallas as pl
from jax.experimental.pallas import tpu as pltpu
from jax.experimental.pallas import tpu_sc as plsc

B, R, W_CNT, D = 4096, 100, 200, 128

NC, NS, L = 2, 16, 16
NW = NC * NS
ROWS_PW = B // NW
G_PW = ROWS_PW // L
WORDS_PW = G_PW * W_CNT * L
TAB_PW = ROWS_PW * R
UNROLL = 4


def _sc_counts_body(wt_hbm, m_hbm, out_hbm, wt_v, m_v, tab_v):
    wid = lax.axis_index("s") * NC + lax.axis_index("c")
    pltpu.sync_copy(wt_hbm.at[pl.ds(wid * WORDS_PW, WORDS_PW)], wt_v)
    pltpu.sync_copy(m_hbm.at[pl.ds(wid * WORDS_PW, WORDS_PW)], m_v)

    zeros = jnp.zeros((L,), jnp.float32)

    def zbody(i, _):
        tab_v[pl.ds(i * L, L)] = zeros
        return 0

    lax.fori_loop(0, TAB_PW // L, zbody, 0)

    lane = lax.iota(jnp.int32, L)
    for g in range(G_PW):
        row_base = (g * L + lane) * R

        def jbody(j, _, g=g, row_base=row_base):
            for u in range(UNROLL):
                k = (g * W_CNT + j * UNROLL + u) * L
                idx = wt_v[pl.ds(k, L)] + row_base
                val = m_v[pl.ds(k, L)]
                plsc.addupdate_scatter(tab_v, [idx], val)
            return 0

        lax.fori_loop(0, W_CNT // UNROLL, jbody, 0)

    pltpu.sync_copy(tab_v, out_hbm.at[pl.ds(wid * TAB_PW, TAB_PW)])


_sc_counts = pl.kernel(
    _sc_counts_body,
    out_type=jax.ShapeDtypeStruct((B * R,), jnp.float32),
    mesh=plsc.VectorSubcoreMesh(core_axis_name="c", subcore_axis_name="s"),
    scratch_types=[
        pltpu.VMEM((WORDS_PW,), jnp.int32),
        pltpu.VMEM((WORDS_PW,), jnp.float32),
        pltpu.VMEM((TAB_PW,), jnp.float32),
    ],
    compiler_params=pltpu.CompilerParams(needs_layout_passes=False),
)

BB = 128
JC = 8


def _tc_body(counts_ref, enc_ref, w_ref, out_ref):
    counts = counts_ref[...]

    pooled = jnp.zeros((BB, D), jnp.float32)
    for c in range(pl.cdiv(R, JC)):
        lo, hi = c * JC, min((c + 1) * JC, R)
        encc = enc_ref[:, lo:hi, :]
        cc = counts[:, lo:hi]
        pooled = pooled + (encc * cc[:, :, None]).sum(axis=1)

    out_ref[...] = jnp.dot(pooled, w_ref[...].T,
                           preferred_element_type=jnp.float32)


@jax.jit
def kernel(encoded_row, wafer_types, loadport_mask, W):
    wt = wafer_types.astype(jnp.int32)
    wt_t = wt.reshape(B // L, L, W_CNT).transpose(0, 2, 1).reshape(-1)
    m_t = (loadport_mask.astype(jnp.float32)
           .reshape(B // L, L, W_CNT).transpose(0, 2, 1).reshape(-1))

    counts = _sc_counts(wt_t, m_t).reshape(B, R)

    grid = (B // BB,)
    return pl.pallas_call(
        _tc_body,
        grid=grid,
        in_specs=[
            pl.BlockSpec((BB, R), lambda i: (i, 0)),
            pl.BlockSpec((BB, R, D), lambda i: (i, 0, 0)),
            pl.BlockSpec((D, D), lambda i: (0, 0)),
        ],
        out_specs=pl.BlockSpec((BB, D), lambda i: (i, 0)),
        out_shape=jax.ShapeDtypeStruct((B, D), jnp.float32),
    )(counts, encoded_row, W)


# speedup vs baseline: 4.9780x; 2.8757x over previous
import functools

import jax
import jax.numpy as jnp
from jax import lax
from jax.experimental import pallas as pl
from jax.experimental.pallas import tpu as pltpu
from jax.experimental.pallas import tpu_sc as plsc

B, R, W_CNT, D = 4096, 100, 200, 128

NC, NS, L = 2, 16, 16
NW = NC * NS
ROWS_PW = B // NW
G_PW = ROWS_PW // L
UNROLL = 4


def _sc_counts_body(wt_hbm, m_hbm, out_hbm, wt_v, m_v, tab_v):
    wid = lax.axis_index("s") * NC + lax.axis_index("c")
    col0 = wid * ROWS_PW
    pltpu.sync_copy(wt_hbm.at[:, pl.ds(col0, ROWS_PW)], wt_v)
    pltpu.sync_copy(m_hbm.at[:, pl.ds(col0, ROWS_PW)], m_v)

    zeros = jnp.zeros((L,), jnp.float32)

    def zbody(i, _):
        for u in range(G_PW):
            tab_v[i, pl.ds(u * L, L)] = zeros
        return 0

    lax.fori_loop(0, R, zbody, 0)

    lane = lax.iota(jnp.int32, L)
    for g in range(G_PW):
        col_base = g * L + lane

        def jbody(j, _, col_base=col_base, g=g):
            for u in range(UNROLL):
                jj = j * UNROLL + u
                idx_r = wt_v[jj, pl.ds(g * L, L)]
                val = m_v[jj, pl.ds(g * L, L)]
                plsc.addupdate_scatter(tab_v, [idx_r, col_base], val)
            return 0

        lax.fori_loop(0, W_CNT // UNROLL, jbody, 0)

    pltpu.sync_copy(tab_v, out_hbm.at[:, pl.ds(col0, ROWS_PW)])


_sc_counts = pl.kernel(
    _sc_counts_body,
    out_type=jax.ShapeDtypeStruct((R, B), jnp.float32),
    mesh=plsc.VectorSubcoreMesh(core_axis_name="c", subcore_axis_name="s"),
    scratch_types=[
        pltpu.VMEM((W_CNT, ROWS_PW), jnp.int32),
        pltpu.VMEM((W_CNT, ROWS_PW), jnp.float32),
        pltpu.VMEM((R, ROWS_PW), jnp.float32),
    ],
    compiler_params=pltpu.CompilerParams(needs_layout_passes=False),
)

BB = 128
JC = 8


def _tc_body(counts_ref, enc_ref, w_ref, out_ref):
    pooled = jnp.zeros((BB, D), jnp.float32)
    for c in range(pl.cdiv(R, JC)):
        lo, hi = c * JC, min((c + 1) * JC, R)
        encc = enc_ref[lo:hi, :, :]
        ccc = counts_ref[lo:hi, :]
        pooled = pooled + (encc * ccc[:, :, None]).sum(axis=0)

    out_ref[...] = jnp.dot(pooled, w_ref[...].T,
                           preferred_element_type=jnp.float32)


@jax.jit
def kernel(encoded_row, wafer_types, loadport_mask, W):
    wt_t = wafer_types.astype(jnp.int32).T
    m_t = loadport_mask.astype(jnp.float32).T
    counts_t = _sc_counts(wt_t, m_t)
    enc_t = encoded_row.transpose(1, 0, 2)

    grid = (B // BB,)
    return pl.pallas_call(
        _tc_body,
        grid=grid,
        in_specs=[
            pl.BlockSpec((R, BB), lambda i: (0, i)),
            pl.BlockSpec((R, BB, D), lambda i: (0, i, 0)),
            pl.BlockSpec((D, D), lambda i: (0, 0)),
        ],
        out_specs=pl.BlockSpec((BB, D), lambda i: (i, 0)),
        out_shape=jax.ShapeDtypeStruct((B, D), jnp.float32),
    )(counts_t, enc_t, W)


# speedup vs baseline: 5.3104x; 1.0668x over previous
import functools

import jax
import jax.numpy as jnp
from jax import lax
from jax.experimental import pallas as pl
from jax.experimental.pallas import tpu as pltpu
from jax.experimental.pallas import tpu_sc as plsc

B, R, W_CNT, D = 4096, 100, 200, 128

NC, NS, L = 2, 16, 16
NW = NC * NS
ROWS_PW = B // NW
G_PW = ROWS_PW // L
UNROLL = 8


def _sc_counts_body(wt_hbm, m_hbm, out_hbm, wt_v, m_v, tab_v, sem_wt, sem_m):
    wid = lax.axis_index("s") * NC + lax.axis_index("c")
    col0 = wid * ROWS_PW
    cp_wt = pltpu.async_copy(wt_hbm.at[:, pl.ds(col0, ROWS_PW)], wt_v, sem_wt)
    cp_m = pltpu.async_copy(m_hbm.at[:, pl.ds(col0, ROWS_PW)], m_v, sem_m)

    zeros = jnp.zeros((L,), jnp.float32)

    def zbody(i, _):
        for u in range(G_PW):
            tab_v[i, pl.ds(u * L, L)] = zeros
        return 0

    lax.fori_loop(0, R, zbody, 0)
    cp_wt.wait()
    cp_m.wait()

    lane = lax.iota(jnp.int32, L)
    for g in range(G_PW):
        col_base = g * L + lane

        def jbody(j, _, col_base=col_base, g=g):
            for u in range(UNROLL):
                jj = j * UNROLL + u
                idx_r = wt_v[jj, pl.ds(g * L, L)]
                val = m_v[jj, pl.ds(g * L, L)]
                plsc.addupdate_scatter(tab_v, [idx_r, col_base], val)
            return 0

        lax.fori_loop(0, W_CNT // UNROLL, jbody, 0)

    pltpu.sync_copy(tab_v, out_hbm.at[:, pl.ds(col0, ROWS_PW)])


_sc_counts = pl.kernel(
    _sc_counts_body,
    out_type=jax.ShapeDtypeStruct((R, B), jnp.float32),
    mesh=plsc.VectorSubcoreMesh(core_axis_name="c", subcore_axis_name="s"),
    scratch_types=[
        pltpu.VMEM((W_CNT, ROWS_PW), jnp.int32),
        pltpu.VMEM((W_CNT, ROWS_PW), jnp.float32),
        pltpu.VMEM((R, ROWS_PW), jnp.float32),
        pltpu.SemaphoreType.DMA,
        pltpu.SemaphoreType.DMA,
    ],
    compiler_params=pltpu.CompilerParams(needs_layout_passes=False),
)

BB = 256
JC = 8


def _tc_body(counts_ref, enc_ref, w_ref, out_ref):
    pooled = jnp.zeros((BB, D), jnp.float32)
    for c in range(pl.cdiv(R, JC)):
        lo, hi = c * JC, min((c + 1) * JC, R)
        encc = enc_ref[lo:hi, :, :]
        ccc = counts_ref[lo:hi, :]
        pooled = pooled + (encc * ccc[:, :, None]).sum(axis=0)

    out_ref[...] = jnp.dot(pooled, w_ref[...].T,
                           preferred_element_type=jnp.float32)


@jax.jit
def kernel(encoded_row, wafer_types, loadport_mask, W):
    wt_t = wafer_types.astype(jnp.int32).T
    m_t = loadport_mask.astype(jnp.float32).T
    counts_t = _sc_counts(wt_t, m_t)
    enc_t = encoded_row.transpose(1, 0, 2)

    grid = (B // BB,)
    return pl.pallas_call(
        _tc_body,
        grid=grid,
        in_specs=[
            pl.BlockSpec((R, BB), lambda i: (0, i)),
            pl.BlockSpec((R, BB, D), lambda i: (0, i, 0)),
            pl.BlockSpec((D, D), lambda i: (0, 0)),
        ],
        out_specs=pl.BlockSpec((BB, D), lambda i: (i, 0)),
        out_shape=jax.ShapeDtypeStruct((B, D), jnp.float32),
    )(counts_t, enc_t, W)
